# baseline (speedup 1.0000x reference)
import functools

import jax
import jax.numpy as jnp
from jax import lax
from jax.experimental import pallas as pl
from jax.experimental.pallas import tpu as pltpu, tpu_sc as plsc

_C = 32


def _sc_gather_t(B: int, D: int, N: int):
    info = plsc.get_sparse_core_info()
    NC, NS = info.num_cores, info.num_subcores
    NW = NC * NS
    assert B % (8 * NW) == 0 and D == 64
    b_per_w = B // NW
    n_chunks = b_per_w // _C
    assert n_chunks * _C == b_per_w and n_chunks % 2 == 0
    mesh = plsc.VectorSubcoreMesh(core_axis_name="c", subcore_axis_name="s")

    @functools.partial(
        pl.kernel,
        mesh=mesh,
        out_type=jax.ShapeDtypeStruct((8, 8, B), jnp.float32),
        compiler_params=pltpu.CompilerParams(needs_layout_passes=False),
        scratch_types=[
            pltpu.VMEM((b_per_w,), jnp.int32),
            pltpu.VMEM((8, 8, _C * 16), jnp.float32),
            pltpu.VMEM((8, 8, _C * 16), jnp.float32),
            pltpu.VMEM((8, 8, b_per_w), jnp.float32),
            pltpu.SemaphoreType.DMA,
            pltpu.SemaphoreType.DMA,
        ],
    )
    def gather_kernel(idx_hbm, table_hbm, out_hbm, idx_v, st0_v, st1_v, ob_v,
                      sem0, sem1):
        wid = lax.axis_index("s") * NC + lax.axis_index("c")
        base = wid * b_per_w
        pltpu.sync_copy(idx_hbm.at[pl.ds(base, b_per_w)], idx_v)

        lane = lax.iota(jnp.int32, 16)
        groups = tuple(((lane + 16 * g) >> 3, lane & 7) for g in range(4))

        def fire(k, st, sem):
            def grp(g):
                v = idx_v[pl.ds(k * _C + g * 16, 16)]
                vb = jax.lax.shift_right_logical(v, 4)
                for l in range(16):
                    pltpu.async_copy(
                        table_hbm.at[:, :, pl.ds(vb[l] * 16, 16)],
                        st.at[:, :, pl.ds((g * 16 + l) * 16, 16)],
                        sem,
                    )
            for g in range(_C // 16):
                grp(g)

        def drain(st, sem):
            pltpu.make_async_copy(
                table_hbm.at[:, :, pl.ds(0, _C * 16)], st, sem
            ).wait()

        def extract(k, st):
            def grp(g):
                v = idx_v[pl.ds(k * _C + g * 16, 16)]
                vc = jax.lax.bitwise_and(v, 15)
                for l in range(16):
                    j = g * 16 + l
                    src_lane = jnp.full((16,), j * 16, jnp.int32) + vc[l]
                    dst_lane = jnp.full((16,), k * _C + j, jnp.int32)
                    for tjv, sv in groups:
                        val = plsc.load_gather(st, [tjv, sv, src_lane])
                        plsc.store_scatter(ob_v, [tjv, sv, dst_lane], val)
            for g in range(_C // 16):
                grp(g)

        fire(0, st0_v, sem0)

        def pair_body(p):
            k0 = p * 2
            fire(k0 + 1, st1_v, sem1)
            drain(st0_v, sem0)
            extract(k0, st0_v)

            @pl.when(p < n_chunks // 2 - 1)
            def _():
                fire(k0 + 2, st0_v, sem0)

            drain(st1_v, sem1)
            extract(k0 + 1, st1_v)

        pl.loop(0, n_chunks // 2)(pair_body)
        pltpu.sync_copy(ob_v, out_hbm.at[:, :, pl.ds(base, b_per_w)])

    return gather_kernel


def kernel(batch_ids, latents):
    B = batch_ids.shape[0]
    N, D = latents.shape
    idx = batch_ids.astype(jnp.int32)
    table_t = latents.T.reshape(8, 8, N)
    out_t = _sc_gather_t(B, D, N)(idx, table_t)
    return out_t.reshape(D, B).T.reshape(B, 1, 1, D)


# speedup vs baseline: 1.4432x; 1.4432x over previous
import functools

import jax
import jax.numpy as jnp
from jax import lax
from jax.experimental import pallas as pl
from jax.experimental.pallas import tpu as pltpu, tpu_sc as plsc

_C = 64


def _sc_gather_t(B: int, D: int, N: int):
    info = plsc.get_sparse_core_info()
    NC, NS = info.num_cores, info.num_subcores
    NW = NC * NS
    assert B % (8 * NW) == 0 and D == 64
    b_per_w = B // NW
    n_chunks = b_per_w // _C
    assert n_chunks * _C == b_per_w
    mesh = plsc.VectorSubcoreMesh(core_axis_name="c", subcore_axis_name="s")

    @functools.partial(
        pl.kernel,
        mesh=mesh,
        out_type=jax.ShapeDtypeStruct((8, 8, B), jnp.float32),
        compiler_params=pltpu.CompilerParams(needs_layout_passes=False),
        scratch_types=[
            pltpu.VMEM((b_per_w,), jnp.int32),
            pltpu.VMEM((8, 8, _C * 16), jnp.float32),
            pltpu.VMEM((8, 8, b_per_w), jnp.float32),
            pltpu.SemaphoreType.DMA,
        ],
    )
    def gather_kernel(idx_hbm, table_hbm, out_hbm, idx_v, st_v, ob_v, sem):
        wid = lax.axis_index("s") * NC + lax.axis_index("c")
        base = wid * b_per_w
        pltpu.sync_copy(idx_hbm.at[pl.ds(base, b_per_w)], idx_v)

        lane = lax.iota(jnp.int32, 16)

        def chunk_body(k):
            def fire(g):
                v = idx_v[pl.ds(k * _C + g * 16, 16)]
                vb = jax.lax.shift_right_logical(v, 4)
                for l in range(16):
                    pltpu.async_copy(
                        table_hbm.at[:, :, pl.ds(vb[l] * 16, 16)],
                        st_v.at[:, :, pl.ds((g * 16 + l) * 16, 16)],
                        sem,
                    )

            pl.loop(0, _C // 16)(fire)
            pltpu.make_async_copy(
                table_hbm.at[:, :, pl.ds(0, _C * 16)], st_v, sem
            ).wait()

            def extract(g):
                v = idx_v[pl.ds(k * _C + g * 16, 16)]
                src_lane = (lane + g * 16) * 16 + jax.lax.bitwise_and(v, 15)
                for tj in range(8):
                    for s in range(8):
                        val = plsc.load_gather(
                            st_v,
                            [jnp.full((16,), tj, jnp.int32),
                             jnp.full((16,), s, jnp.int32),
                             src_lane],
                        )
                        ob_v[tj, s, pl.ds(k * _C + g * 16, 16)] = val

            pl.loop(0, _C // 16)(extract)

        pl.loop(0, n_chunks)(chunk_body)
        pltpu.sync_copy(ob_v, out_hbm.at[:, :, pl.ds(base, b_per_w)])

    return gather_kernel


def kernel(batch_ids, latents):
    B = batch_ids.shape[0]
    N, D = latents.shape
    idx = batch_ids.astype(jnp.int32)
    table_t = latents.T.reshape(8, 8, N)
    out_t = _sc_gather_t(B, D, N)(idx, table_t)
    return out_t.reshape(D, B).T.reshape(B, 1, 1, D)
